# Initial kernel scaffold; baseline (speedup 1.0000x reference)
import jax
import jax.numpy as jnp
from jax.experimental import pallas as pl


def kernel(input_, embed_weight, W_proj, b_proj):
    raise NotImplementedError("write your pallas kernel here")


# speedup vs baseline: 2.4056x; 2.4056x over previous
import functools

import jax
import jax.numpy as jnp
from jax import lax
from jax.experimental import pallas as pl
from jax.experimental.pallas import tpu as pltpu
from jax.experimental.pallas import tpu_sc as plsc

_NC = 2
_NS = 16
_NW = _NC * _NS

_LANES = 16


@functools.lru_cache(maxsize=None)
def _make_bag_mean(batch, hist, edim, num_emb):
    bags_per_w = batch // _NW
    bags_per_chunk = 8
    chunks_per_w = bags_per_w // bags_per_chunk
    num_chunks = batch // bags_per_chunk
    nseg = (bags_per_chunk * hist) // 100
    seg = 100
    cvecs = edim // _LANES

    mesh = plsc.VectorSubcoreMesh(core_axis_name="c", subcore_axis_name="s")

    @functools.partial(
        pl.kernel,
        out_type=jax.ShapeDtypeStruct((batch, edim), jnp.float32),
        mesh=mesh,
        scratch_types=[
            pltpu.VMEM((nseg, seg), jnp.int32),
            pltpu.VMEM((nseg, seg, edim), jnp.float32),
            pltpu.VMEM((bags_per_w, edim), jnp.float32),
            pltpu.SemaphoreType.DMA,
        ],
        compiler_params=pltpu.CompilerParams(use_tc_tiling_on_sc=False),
    )
    def bag_mean(idx_hbm, table_hbm, out_hbm, idx_v, rows_v, out_v, sem):
        wid = lax.axis_index("s") * _NC + lax.axis_index("c")
        chunk0 = wid * chunks_per_w

        def chunk_body(g, carry):
            pltpu.sync_copy(idx_hbm.at[chunk0 + g], idx_v)
            for j in range(nseg):
                pltpu.async_copy(table_hbm.at[idx_v.at[j]], rows_v.at[j], sem)
            for j in range(nseg):
                pltpu.make_async_copy(
                    table_hbm.at[idx_v.at[j]], rows_v.at[j], sem).wait()
            for b in range(bags_per_chunk):
                j, r0 = divmod(b * hist, seg)
                unroll = 5
                def rbody(r, accs, j=j, r0=r0):
                    a = list(accs)
                    base = r0 + r * unroll
                    for u in range(unroll):
                        for c in range(cvecs):
                            a[c] = a[c] + rows_v[j, base + u,
                                                 pl.ds(c * _LANES, _LANES)]
                    return tuple(a)
                accs = lax.fori_loop(
                    0, hist // unroll, rbody,
                    tuple(jnp.zeros((_LANES,), jnp.float32)
                          for _ in range(cvecs)))
                orow = g * bags_per_chunk + b
                for c in range(cvecs):
                    out_v[orow, pl.ds(c * _LANES, _LANES)] = (
                        accs[c] * (1.0 / hist))
            return carry

        lax.fori_loop(0, chunks_per_w, chunk_body, 0)
        pltpu.sync_copy(out_v, out_hbm.at[pl.ds(wid * bags_per_w, bags_per_w)])

    return bag_mean


def _proj_body(p_ref, w_ref, b_ref, o_ref):
    o_ref[...] = lax.dot_general(
        p_ref[...], w_ref[...], (((1,), (1,)), ((), ())),
        preferred_element_type=jnp.float32) + b_ref[...]


def _project(pooled, w_proj, b_proj):
    batch, edim = pooled.shape
    odim = w_proj.shape[0]
    blk = 1024
    return pl.pallas_call(
        _proj_body,
        grid=(batch // blk,),
        in_specs=[
            pl.BlockSpec((blk, edim), lambda i: (i, 0)),
            pl.BlockSpec((odim, edim), lambda i: (0, 0)),
            pl.BlockSpec((1, odim), lambda i: (0, 0)),
        ],
        out_specs=pl.BlockSpec((blk, odim), lambda i: (i, 0)),
        out_shape=jax.ShapeDtypeStruct((batch, odim), jnp.float32),
    )(pooled, w_proj, b_proj.reshape(1, odim))


def kernel(input_, embed_weight, W_proj, b_proj):
    batch, hist = input_.shape
    num_emb, edim = embed_weight.shape
    bags_per_chunk = 8
    num_chunks = batch // bags_per_chunk
    nseg = (bags_per_chunk * hist) // 100
    idx3d = input_.reshape(num_chunks, nseg, 100)
    pooled = _make_bag_mean(batch, hist, edim, num_emb)(idx3d, embed_weight)
    return _project(pooled, W_proj, b_proj)
